# baseline (speedup 1.0000x reference)
import jax
import jax.numpy as jnp
from jax.experimental import pallas as pl
from jax.experimental.pallas import tpu as pltpu
from jax.experimental.pallas import tpu_sc as plsc

_C = 4
_L = 16


def _sc_body_factory(B, C, D):
    def body(pos_v, x_v, o_v):
        @pl.loop(0, C)
        def _(r):
            @pl.loop(0, D, step=_L)
            def _(j):
                p = pos_v.at[r, pl.ds(j, _L)][...]
                for b in range(B):
                    o_v.at[b, r, pl.ds(j, _L)][...] = (
                        x_v.at[b, r, pl.ds(j, _L)][...] + p
                    )
    return body


@jax.jit
def kernel(x, token_embedding):
    B, N, D = x.shape
    mesh = plsc.VectorSubcoreMesh(core_axis_name="c", subcore_axis_name="s")

    @pl.kernel(out_type=jax.ShapeDtypeStruct((B, N, D), x.dtype), mesh=mesh)
    def sc_add(pos_hbm, x_hbm, o_hbm):
        pltpu.emit_pipeline(
            _sc_body_factory(B, _C, D),
            grid=(N // _C,),
            in_specs=[
                pl.BlockSpec((_C, D), index_map=lambda i: (i, 0)),
                pl.BlockSpec((B, _C, D), index_map=lambda i: (0, i, 0)),
            ],
            out_specs=[pl.BlockSpec((B, _C, D), index_map=lambda i: (0, i, 0))],
            core_axis_name=("c", "s"),
            dimension_semantics=(pltpu.PARALLEL,),
        )(pos_hbm, x_hbm, o_hbm)

    return sc_add(token_embedding, x)


# speedup vs baseline: 1.0968x; 1.0968x over previous
import jax
import jax.numpy as jnp
from jax.experimental import pallas as pl
from jax.experimental.pallas import tpu as pltpu
from jax.experimental.pallas import tpu_sc as plsc

_C = 4
_L = 16


def _sc_body_factory(B, C, D):
    UNROLL = 8

    def body(pos_v, x_v, o_v):
        @pl.loop(0, C)
        def _(r):
            @pl.loop(0, D, step=_L * UNROLL)
            def _(j):
                for u in range(UNROLL):
                    c = pl.ds(j + u * _L, _L)
                    p = pos_v.at[r, c][...]
                    for b in range(B):
                        o_v.at[b, r, c][...] = x_v.at[b, r, c][...] + p
    return body


@jax.jit
def kernel(x, token_embedding):
    B, N, D = x.shape
    mesh = plsc.VectorSubcoreMesh(core_axis_name="c", subcore_axis_name="s")

    @pl.kernel(out_type=jax.ShapeDtypeStruct((B, N, D), x.dtype), mesh=mesh)
    def sc_add(pos_hbm, x_hbm, o_hbm):
        pltpu.emit_pipeline(
            _sc_body_factory(B, _C, D),
            grid=(N // _C,),
            in_specs=[
                pl.BlockSpec((_C, D), index_map=lambda i: (i, 0)),
                pl.BlockSpec((B, _C, D), index_map=lambda i: (0, i, 0)),
            ],
            out_specs=[pl.BlockSpec((B, _C, D), index_map=lambda i: (0, i, 0))],
            core_axis_name=("c", "s"),
            dimension_semantics=(pltpu.PARALLEL,),
        )(pos_hbm, x_hbm, o_hbm)

    return sc_add(token_embedding, x)


# speedup vs baseline: 4.1734x; 3.8049x over previous
import jax
import jax.numpy as jnp
from jax.experimental import pallas as pl

_BN = 2048


def _add_body(x_ref, pos_ref, o_ref):
    o_ref[0] = x_ref[0] + pos_ref[...]


@jax.jit
def kernel(x, token_embedding):
    B, N, D = x.shape
    return pl.pallas_call(
        _add_body,
        grid=(N // _BN, B),
        in_specs=[
            pl.BlockSpec((1, _BN, D), lambda n, b: (b, n, 0)),
            pl.BlockSpec((_BN, D), lambda n, b: (n, 0)),
        ],
        out_specs=pl.BlockSpec((1, _BN, D), lambda n, b: (b, n, 0)),
        out_shape=jax.ShapeDtypeStruct((B, N, D), x.dtype),
    )(x, token_embedding)
